# Initial kernel scaffold; baseline (speedup 1.0000x reference)
import jax
import jax.numpy as jnp
from jax.experimental import pallas as pl


def kernel(x, edge_index, edge_attr, batch, emb0, emb1, emb2, emb3, emb4, emb5, emb6, conv_w, conv_b, w1, b1, g1, be1, w2, b2, g2, be2, w3, b3):
    raise NotImplementedError("write your pallas kernel here")


# speedup vs baseline: 1.0243x; 1.0243x over previous
import functools

import jax
import jax.numpy as jnp
import numpy as np
from jax.experimental import pallas as pl
from jax.experimental.pallas import tpu as pltpu

N_NODES_C = 10000
HIDDEN_C = 112
NUM_GRAPHS_C = 128
EPS_C = 1e-5
Q_C = 0.25

ROW_BLK = 1000


def _conv_layer_body(t0re_ref, t1re_ref, t2re_ref, t0im_ref, t1im_ref, t2im_ref,
                     w_ref, b_ref, outre_ref, outim_ref):
    w0 = w_ref[0]
    w1 = w_ref[1]
    w2 = w_ref[2]
    b = b_ref[...]
    outre = (jnp.dot(t0re_ref[...], w0, preferred_element_type=jnp.float32)
             + jnp.dot(t1re_ref[...], w1, preferred_element_type=jnp.float32)
             + jnp.dot(t2re_ref[...], w2, preferred_element_type=jnp.float32)
             + b[None, :])
    outim = (jnp.dot(t0im_ref[...], w0, preferred_element_type=jnp.float32)
             + jnp.dot(t1im_ref[...], w1, preferred_element_type=jnp.float32)
             + jnp.dot(t2im_ref[...], w2, preferred_element_type=jnp.float32)
             + b[None, :])
    mask = (outre >= 0.0).astype(jnp.float32)
    outre_ref[...] = mask * outre
    outim_ref[...] = mask * outim


def _conv_layer(t0re, t1re, t2re, t0im, t1im, t2im, w, b):
    n = t0re.shape[0]
    h = t0re.shape[1]
    grid = n // ROW_BLK
    blk = pl.BlockSpec((ROW_BLK, h), lambda i: (i, 0))
    full_w = pl.BlockSpec((3, h, h), lambda i: (0, 0, 0))
    full_b = pl.BlockSpec((h,), lambda i: (0,))
    return pl.pallas_call(
        _conv_layer_body,
        grid=(grid,),
        in_specs=[blk, blk, blk, blk, blk, blk, full_w, full_b],
        out_specs=[blk, blk],
        out_shape=[jax.ShapeDtypeStruct((n, h), jnp.float32),
                   jax.ShapeDtypeStruct((n, h), jnp.float32)],
    )(t0re, t1re, t2re, t0im, t1im, t2im, w, b)


def _head_body(hre_ref, him_ref, batch_ref, w1_ref, b1_ref, g1_ref, be1_ref,
               w2_ref, b2_ref, g2_ref, be2_ref, w3_ref, b3_ref, out_ref, acc_ref):
    i = pl.program_id(0)
    ids = batch_ref[...]
    onehot = (ids == jax.lax.broadcasted_iota(jnp.int32, (1, NUM_GRAPHS_C), 1)
              ).astype(jnp.float32)
    hcat = jnp.concatenate([hre_ref[...], him_ref[...]], axis=1)
    part = jax.lax.dot_general(onehot, hcat, (((0,), (0,)), ((), ())),
                               preferred_element_type=jnp.float32,
                               precision=jax.lax.Precision.HIGHEST)

    @pl.when(i == 0)
    def _init():
        acc_ref[...] = jnp.zeros_like(acc_ref)

    acc_ref[...] += part

    @pl.when(i == pl.num_programs(0) - 1)
    def _fin():
        pooled = acc_ref[...]

        def bn(z, g, be):
            mu = jnp.mean(z, axis=0, keepdims=True)
            var = jnp.mean((z - mu) ** 2, axis=0, keepdims=True)
            return (z - mu) / jnp.sqrt(var + EPS_C) * g[None, :] + be[None, :]

        z = jnp.dot(pooled, w1_ref[...], preferred_element_type=jnp.float32) + b1_ref[...][None, :]
        z = jax.nn.relu(bn(z, g1_ref[...], be1_ref[...]))
        z = jnp.dot(z, w2_ref[...], preferred_element_type=jnp.float32) + b2_ref[...][None, :]
        z = jax.nn.relu(bn(z, g2_ref[...], be2_ref[...]))
        out_ref[...] = jnp.dot(z, w3_ref[...], preferred_element_type=jnp.float32) + b3_ref[...][None, :]


def _head(hre, him, batch, w1, b1, g1, be1, w2, b2, g2, be2, w3, b3):
    n, h = hre.shape
    grid = n // ROW_BLK
    blk = pl.BlockSpec((ROW_BLK, h), lambda i: (i, 0))
    bblk = pl.BlockSpec((ROW_BLK, 1), lambda i: (i, 0))
    full = lambda *shape: pl.BlockSpec(shape, lambda i: tuple(0 for _ in shape))
    h2 = w1.shape[1]
    h4 = w2.shape[1]
    return pl.pallas_call(
        _head_body,
        grid=(grid,),
        in_specs=[blk, blk, bblk,
                  full(2 * h, h2), full(h2), full(h2), full(h2),
                  full(h2, h4), full(h4), full(h4), full(h4),
                  full(h4, 1), full(1)],
        out_specs=pl.BlockSpec((NUM_GRAPHS_C, 1), lambda i: (0, 0)),
        out_shape=jax.ShapeDtypeStruct((NUM_GRAPHS_C, 1), jnp.float32),
        scratch_shapes=[pltpu.VMEM((NUM_GRAPHS_C, 2 * h), jnp.float32)],
    )(hre, him, batch.reshape(n, 1), w1, b1, g1, be1, w2, b2, g2, be2, w3, b3)


def _spmv_complex(u_row, u_col, lre, lim_, xre, xim, n):
    xr = xre[u_col]
    xi = xim[u_col]
    mre = lre[:, None] * xr - lim_[:, None] * xi
    mim = lre[:, None] * xi + lim_[:, None] * xr
    yre = jnp.zeros((n, xre.shape[1]), xre.dtype).at[u_row].add(mre)
    yim = jnp.zeros((n, xre.shape[1]), xre.dtype).at[u_row].add(mim)
    return yre, yim


def kernel(x, edge_index, edge_attr, batch, emb0, emb1, emb2, emb3, emb4,
           emb5, emb6, conv_w, conv_b, w1, b1, g1, be1, w2, b2, g2, be2,
           w3, b3):
    n = x.shape[0]
    embs = [emb0, emb1, emb2, emb3, emb4, emb5, emb6]

    row, col = edge_index[0], edge_index[1]
    ids = jnp.concatenate([row * n + col, col * n + row])
    m = ids.shape[0]
    uniq, inv = jnp.unique(ids, return_inverse=True, size=m, fill_value=0)
    u_row = (uniq // n).astype(jnp.int32)
    u_col = (uniq % n).astype(jnp.int32)
    inv = inv.reshape(-1)

    ew = edge_attr[:, 0] + edge_attr[:, 1]
    sym_w = jnp.concatenate([ew, ew]) * 0.5
    theta_w = jnp.concatenate([ew, -ew]) * (2.0 * np.pi * Q_C)
    sym = jnp.zeros((m,), jnp.float32).at[inv].add(sym_w)
    theta = jnp.zeros((m,), jnp.float32).at[inv].add(theta_w)
    deg = jnp.zeros((n,), jnp.float32).at[u_row].add(jnp.abs(sym))
    safe = jnp.where(deg > 0.0, deg, 1.0)
    dinv = jnp.where(deg > 0.0, 1.0 / jnp.sqrt(safe), 0.0)
    norm = dinv[u_row] * sym * dinv[u_col]
    lre = -norm * jnp.cos(theta)
    lim_ = -norm * jnp.sin(theta)

    h = jnp.concatenate([embs[i][x[:, i]] for i in range(7)], axis=1)
    hre, him = h, h

    for l in range(conv_w.shape[0]):
        t0re, t0im = hre, him
        t1re, t1im = _spmv_complex(u_row, u_col, lre, lim_, t0re, t0im, n)
        t2re, t2im = _spmv_complex(u_row, u_col, lre, lim_, t1re, t1im, n)
        t2re = 2.0 * t2re - t0re
        t2im = 2.0 * t2im - t0im
        hre, him = _conv_layer(t0re, t1re, t2re, t0im, t1im, t2im,
                               conv_w[l], conv_b[l])

    return _head(hre, him, batch, w1, b1, g1, be1, w2, b2, g2, be2, w3, b3)


# speedup vs baseline: 1.2157x; 1.1868x over previous
import functools

import jax
import jax.numpy as jnp
import numpy as np
from jax import lax
from jax.experimental import pallas as pl
from jax.experimental.pallas import tpu as pltpu
from jax.experimental.pallas import tpu_sc as plsc

N_NODES_C = 10000
HIDDEN_C = 112
NUM_GRAPHS_C = 128
EPS_C = 1e-5
Q_C = 0.25

ROW_BLK = 1000
HPAD = 128


def _conv_layer_body(t0re_ref, t1re_ref, t2re_ref, t0im_ref, t1im_ref, t2im_ref,
                     w_ref, b_ref, outre_ref, outim_ref):
    w0 = w_ref[0]
    w1 = w_ref[1]
    w2 = w_ref[2]
    b = b_ref[...]
    outre = (jnp.dot(t0re_ref[...], w0, preferred_element_type=jnp.float32)
             + jnp.dot(t1re_ref[...], w1, preferred_element_type=jnp.float32)
             + jnp.dot(t2re_ref[...], w2, preferred_element_type=jnp.float32)
             + b[None, :])
    outim = (jnp.dot(t0im_ref[...], w0, preferred_element_type=jnp.float32)
             + jnp.dot(t1im_ref[...], w1, preferred_element_type=jnp.float32)
             + jnp.dot(t2im_ref[...], w2, preferred_element_type=jnp.float32)
             + b[None, :])
    mask = (outre >= 0.0).astype(jnp.float32)
    outre_ref[...] = mask * outre
    outim_ref[...] = mask * outim


def _conv_layer(t0re, t1re, t2re, t0im, t1im, t2im, w, b):
    n = t0re.shape[0]
    h = t0re.shape[1]
    grid = n // ROW_BLK
    blk = pl.BlockSpec((ROW_BLK, h), lambda i: (i, 0))
    full_w = pl.BlockSpec((3, h, h), lambda i: (0, 0, 0))
    full_b = pl.BlockSpec((h,), lambda i: (0,))
    return pl.pallas_call(
        _conv_layer_body,
        grid=(grid,),
        in_specs=[blk, blk, blk, blk, blk, blk, full_w, full_b],
        out_specs=[blk, blk],
        out_shape=[jax.ShapeDtypeStruct((n, h), jnp.float32),
                   jax.ShapeDtypeStruct((n, h), jnp.float32)],
    )(t0re, t1re, t2re, t0im, t1im, t2im, w, b)


def _head_body(hre_ref, him_ref, batch_ref, w1_ref, b1_ref, g1_ref, be1_ref,
               w2_ref, b2_ref, g2_ref, be2_ref, w3_ref, b3_ref, out_ref, acc_ref):
    i = pl.program_id(0)
    ids = batch_ref[...]
    onehot = (ids == jax.lax.broadcasted_iota(jnp.int32, (1, NUM_GRAPHS_C), 1)
              ).astype(jnp.float32)
    hcat = jnp.concatenate([hre_ref[...], him_ref[...]], axis=1)
    part = jax.lax.dot_general(onehot, hcat, (((0,), (0,)), ((), ())),
                               preferred_element_type=jnp.float32,
                               precision=jax.lax.Precision.HIGHEST)

    @pl.when(i == 0)
    def _init():
        acc_ref[...] = jnp.zeros_like(acc_ref)

    acc_ref[...] += part

    @pl.when(i == pl.num_programs(0) - 1)
    def _fin():
        pooled = acc_ref[...]

        def bn(z, g, be):
            mu = jnp.mean(z, axis=0, keepdims=True)
            var = jnp.mean((z - mu) ** 2, axis=0, keepdims=True)
            return (z - mu) / jnp.sqrt(var + EPS_C) * g[None, :] + be[None, :]

        z = jnp.dot(pooled, w1_ref[...], preferred_element_type=jnp.float32) + b1_ref[...][None, :]
        z = jax.nn.relu(bn(z, g1_ref[...], be1_ref[...]))
        z = jnp.dot(z, w2_ref[...], preferred_element_type=jnp.float32) + b2_ref[...][None, :]
        z = jax.nn.relu(bn(z, g2_ref[...], be2_ref[...]))
        out_ref[...] = jnp.dot(z, w3_ref[...], preferred_element_type=jnp.float32) + b3_ref[...][None, :]


def _head(hre, him, batch, w1, b1, g1, be1, w2, b2, g2, be2, w3, b3):
    n, h = hre.shape
    grid = n // ROW_BLK
    blk = pl.BlockSpec((ROW_BLK, h), lambda i: (i, 0))
    bblk = pl.BlockSpec((ROW_BLK, 1), lambda i: (i, 0))
    full = lambda *shape: pl.BlockSpec(shape, lambda i: tuple(0 for _ in shape))
    h2 = w1.shape[1]
    h4 = w2.shape[1]
    return pl.pallas_call(
        _head_body,
        grid=(grid,),
        in_specs=[blk, blk, bblk,
                  full(2 * h, h2), full(h2), full(h2), full(h2),
                  full(h2, h4), full(h4), full(h4), full(h4),
                  full(h4, 1), full(1)],
        out_specs=pl.BlockSpec((NUM_GRAPHS_C, 1), lambda i: (0, 0)),
        out_shape=jax.ShapeDtypeStruct((NUM_GRAPHS_C, 1), jnp.float32),
        scratch_shapes=[pltpu.VMEM((NUM_GRAPHS_C, 2 * h), jnp.float32)],
    )(hre, him, batch.reshape(n, 1), w1, b1, g1, be1, w2, b2, g2, be2, w3, b3)


def _spmv_complex(u_row, u_col, lre, lim_, xre, xim, n):
    xr = xre[u_col]
    xi = xim[u_col]
    mre = lre[:, None] * xr - lim_[:, None] * xi
    mim = lre[:, None] * xi + lim_[:, None] * xr
    yre = jnp.zeros((n, xre.shape[1]), xre.dtype).at[u_row].add(mre)
    yim = jnp.zeros((n, xre.shape[1]), xre.dtype).at[u_row].add(mim)
    return yre, yim


CHUNK = 128
LANES = 16
NSUB = 16
ZROWS = 80


def _lane_bcast(v, lane):
    idx = jnp.full((LANES, 1), lane, jnp.int32)
    dn = lax.GatherDimensionNumbers(offset_dims=(), collapsed_slice_dims=(0,),
                                    start_index_map=(0,))
    return lax.gather(v, idx, dn, (1,),
                      mode=lax.GatherScatterMode.PROMISE_IN_BOUNDS)


def _spmv_sc(xre, xim, ucol, urow, lre, lim_, n):
    m = ucol.shape[0]
    h = xre.shape[1]
    nchunks = m // CHUNK
    ngroups = h // LANES
    mesh = plsc.VectorSubcoreMesh(core_axis_name="c", subcore_axis_name="s")

    @functools.partial(
        pl.kernel,
        mesh=mesh,
        out_type=[jax.ShapeDtypeStruct((n, h), jnp.float32),
                  jax.ShapeDtypeStruct((n, h), jnp.float32)],
        scratch_types=[
            pltpu.VMEM((CHUNK,), jnp.int32),
            pltpu.VMEM((1, CHUNK), jnp.int32),
            pltpu.VMEM((CHUNK,), jnp.float32),
            pltpu.VMEM((CHUNK,), jnp.float32),
            pltpu.VMEM((CHUNK, h), jnp.float32),
            pltpu.VMEM((CHUNK, h), jnp.float32),
            pltpu.VMEM_SHARED((n, h), jnp.float32),
        ],
    )
    def k(xre_hbm, xim_hbm, ucol_hbm, urow_hbm, lre_hbm, lim_hbm,
          yre_hbm, yim_hbm,
          gidx_v, sidx_v, la_v, lb_v, rr_v, ri_v, y_sh):
        core = lax.axis_index("c")
        sub = lax.axis_index("s")
        is_re = core == 0
        cf = lax.convert_element_type(1 - core, jnp.float32)
        cfv = jnp.full((LANES,), cf)
        ofv = 1.0 - cfv

        zv = jnp.zeros((LANES,), jnp.float32)

        def _zrow(r, _):
            for g in range(ngroups):
                rr_v[r, pl.ds(g * LANES, LANES)] = zv
            return 0

        lax.fori_loop(0, CHUNK, _zrow, 0)
        nblk = n // ZROWS
        nround = (nblk + NSUB - 1) // NSUB
        for bb in range(nround):
            blk = sub + bb * NSUB

            @pl.when(blk < nblk)
            def _():
                r0 = pl.multiple_of(blk * ZROWS, ZROWS)
                pltpu.sync_copy(rr_v.at[pl.ds(0, ZROWS)],
                                y_sh.at[pl.ds(r0, ZROWS)])
        plsc.subcore_barrier()

        def _chunk(kk, _):
            c = sub + kk * NSUB

            @pl.when(c < nchunks)
            def _():
                base = pl.multiple_of(c * CHUNK, CHUNK)
                pltpu.sync_copy(ucol_hbm.at[pl.ds(base, CHUNK)], gidx_v)
                pltpu.sync_copy(urow_hbm.at[pl.ds(base, CHUNK)],
                                sidx_v.at[0])
                pltpu.sync_copy(lre_hbm.at[pl.ds(base, CHUNK)], la_v)
                pltpu.sync_copy(lim_hbm.at[pl.ds(base, CHUNK)], lb_v)
                pltpu.sync_copy(xre_hbm.at[gidx_v], rr_v)
                pltpu.sync_copy(xim_hbm.at[gidx_v], ri_v)
                for mb in range(CHUNK // LANES):
                    la = la_v[pl.ds(mb * LANES, LANES)]
                    lb = lb_v[pl.ds(mb * LANES, LANES)]
                    av = la * cfv + lb * ofv
                    bv = la * ofv - lb * cfv
                    for e0 in range(LANES):
                        e = mb * LANES + e0
                        a_bc = _lane_bcast(av, e0)
                        b_bc = _lane_bcast(bv, e0)
                        for g in range(ngroups):
                            xr = rr_v[e, pl.ds(g * LANES, LANES)]
                            xi = ri_v[e, pl.ds(g * LANES, LANES)]
                            rr_v[e, pl.ds(g * LANES, LANES)] = (
                                a_bc * xr + b_bc * xi)
                pltpu.sync_copy(rr_v, y_sh.at[sidx_v.at[0]], add=True)
            return 0

        lax.fori_loop(0, (nchunks + NSUB - 1) // NSUB, _chunk, 0)
        plsc.subcore_barrier()

        for bb in range(nround):
            blk = sub + bb * NSUB

            @pl.when(blk < nblk)
            def _():
                r0 = pl.multiple_of(blk * ZROWS, ZROWS)
                pltpu.sync_copy(y_sh.at[pl.ds(r0, ZROWS)],
                                rr_v.at[pl.ds(0, ZROWS)])

                @pl.when(is_re)
                def _():
                    pltpu.sync_copy(rr_v.at[pl.ds(0, ZROWS)],
                                    yre_hbm.at[pl.ds(r0, ZROWS)])

                @pl.when(jnp.logical_not(is_re))
                def _():
                    pltpu.sync_copy(rr_v.at[pl.ds(0, ZROWS)],
                                    yim_hbm.at[pl.ds(r0, ZROWS)])

    return k(xre, xim, ucol, urow, lre, lim_)


def kernel(x, edge_index, edge_attr, batch, emb0, emb1, emb2, emb3, emb4,
           emb5, emb6, conv_w, conv_b, w1, b1, g1, be1, w2, b2, g2, be2,
           w3, b3):
    n = x.shape[0]
    embs = [emb0, emb1, emb2, emb3, emb4, emb5, emb6]

    row, col = edge_index[0], edge_index[1]
    ids = jnp.concatenate([row * n + col, col * n + row])
    m = ids.shape[0]
    uniq, inv = jnp.unique(ids, return_inverse=True, size=m, fill_value=0)
    u_row = (uniq // n).astype(jnp.int32)
    u_col = (uniq % n).astype(jnp.int32)
    inv = inv.reshape(-1)

    ew = edge_attr[:, 0] + edge_attr[:, 1]
    sym_w = jnp.concatenate([ew, ew]) * 0.5
    theta_w = jnp.concatenate([ew, -ew]) * (2.0 * np.pi * Q_C)
    sym = jnp.zeros((m,), jnp.float32).at[inv].add(sym_w)
    theta = jnp.zeros((m,), jnp.float32).at[inv].add(theta_w)
    deg = jnp.zeros((n,), jnp.float32).at[u_row].add(jnp.abs(sym))
    safe = jnp.where(deg > 0.0, deg, 1.0)
    dinv = jnp.where(deg > 0.0, 1.0 / jnp.sqrt(safe), 0.0)
    norm = dinv[u_row] * sym * dinv[u_col]
    lre = -norm * jnp.cos(theta)
    lim_ = -norm * jnp.sin(theta)

    h = jnp.concatenate([embs[i][x[:, i]] for i in range(7)], axis=1)
    h = jnp.pad(h, ((0, 0), (0, HPAD - HIDDEN_C)))
    hre, him = h, h

    wp = jnp.pad(conv_w, ((0, 0), (0, 0), (0, HPAD - HIDDEN_C), (0, HPAD - HIDDEN_C)))
    bp = jnp.pad(conv_b, ((0, 0), (0, HPAD - HIDDEN_C)))
    w1p = jnp.zeros((2 * HPAD, w1.shape[1]), jnp.float32)
    w1p = w1p.at[:HIDDEN_C].set(w1[:HIDDEN_C])
    w1p = w1p.at[HPAD:HPAD + HIDDEN_C].set(w1[HIDDEN_C:])

    for l in range(conv_w.shape[0]):
        t0re, t0im = hre, him
        t1re, t1im = _spmv_sc(t0re, t0im, u_col, u_row, lre, lim_, n)
        t2re, t2im = _spmv_sc(t1re, t1im, u_col, u_row, lre, lim_, n)
        t2re = 2.0 * t2re - t0re
        t2im = 2.0 * t2im - t0im
        hre, him = _conv_layer(t0re, t1re, t2re, t0im, t1im, t2im,
                               wp[l], bp[l])

    return _head(hre, him, batch, w1p, b1, g1, be1, w2, b2, g2, be2, w3, b3)


# speedup vs baseline: 1.6954x; 1.3946x over previous
import functools

import jax
import jax.numpy as jnp
import numpy as np
from jax import lax
from jax.experimental import pallas as pl
from jax.experimental.pallas import tpu as pltpu
from jax.experimental.pallas import tpu_sc as plsc

N_NODES_C = 10000
HIDDEN_C = 112
NUM_GRAPHS_C = 128
EPS_C = 1e-5
Q_C = 0.25

ROW_BLK = 1000
HPAD = 128


def _conv_layer_body(t0re_ref, t1re_ref, t2re_ref, t0im_ref, t1im_ref, t2im_ref,
                     w_ref, b_ref, outre_ref, outim_ref):
    w0 = w_ref[0]
    w1 = w_ref[1]
    w2 = w_ref[2]
    b = b_ref[...]
    outre = (jnp.dot(t0re_ref[...], w0, preferred_element_type=jnp.float32)
             + jnp.dot(t1re_ref[...], w1, preferred_element_type=jnp.float32)
             + jnp.dot(t2re_ref[...], w2, preferred_element_type=jnp.float32)
             + b[None, :])
    outim = (jnp.dot(t0im_ref[...], w0, preferred_element_type=jnp.float32)
             + jnp.dot(t1im_ref[...], w1, preferred_element_type=jnp.float32)
             + jnp.dot(t2im_ref[...], w2, preferred_element_type=jnp.float32)
             + b[None, :])
    mask = (outre >= 0.0).astype(jnp.float32)
    outre_ref[...] = mask * outre
    outim_ref[...] = mask * outim


def _conv_layer(t0re, t1re, t2re, t0im, t1im, t2im, w, b):
    n = t0re.shape[0]
    h = t0re.shape[1]
    grid = n // ROW_BLK
    blk = pl.BlockSpec((ROW_BLK, h), lambda i: (i, 0))
    full_w = pl.BlockSpec((3, h, h), lambda i: (0, 0, 0))
    full_b = pl.BlockSpec((h,), lambda i: (0,))
    return pl.pallas_call(
        _conv_layer_body,
        grid=(grid,),
        in_specs=[blk, blk, blk, blk, blk, blk, full_w, full_b],
        out_specs=[blk, blk],
        out_shape=[jax.ShapeDtypeStruct((n, h), jnp.float32),
                   jax.ShapeDtypeStruct((n, h), jnp.float32)],
    )(t0re, t1re, t2re, t0im, t1im, t2im, w, b)


def _head_body(hre_ref, him_ref, batch_ref, w1_ref, b1_ref, g1_ref, be1_ref,
               w2_ref, b2_ref, g2_ref, be2_ref, w3_ref, b3_ref, out_ref, acc_ref):
    i = pl.program_id(0)
    ids = batch_ref[...]
    onehot = (ids == jax.lax.broadcasted_iota(jnp.int32, (1, NUM_GRAPHS_C), 1)
              ).astype(jnp.float32)
    hcat = jnp.concatenate([hre_ref[...], him_ref[...]], axis=1)
    part = jax.lax.dot_general(onehot, hcat, (((0,), (0,)), ((), ())),
                               preferred_element_type=jnp.float32,
                               precision=jax.lax.Precision.HIGHEST)

    @pl.when(i == 0)
    def _init():
        acc_ref[...] = jnp.zeros_like(acc_ref)

    acc_ref[...] += part

    @pl.when(i == pl.num_programs(0) - 1)
    def _fin():
        pooled = acc_ref[...]

        def bn(z, g, be):
            mu = jnp.mean(z, axis=0, keepdims=True)
            var = jnp.mean((z - mu) ** 2, axis=0, keepdims=True)
            return (z - mu) / jnp.sqrt(var + EPS_C) * g[None, :] + be[None, :]

        z = jnp.dot(pooled, w1_ref[...], preferred_element_type=jnp.float32) + b1_ref[...][None, :]
        z = jax.nn.relu(bn(z, g1_ref[...], be1_ref[...]))
        z = jnp.dot(z, w2_ref[...], preferred_element_type=jnp.float32) + b2_ref[...][None, :]
        z = jax.nn.relu(bn(z, g2_ref[...], be2_ref[...]))
        out_ref[...] = jnp.dot(z, w3_ref[...], preferred_element_type=jnp.float32) + b3_ref[...][None, :]


def _head(hre, him, batch, w1, b1, g1, be1, w2, b2, g2, be2, w3, b3):
    n, h = hre.shape
    grid = n // ROW_BLK
    blk = pl.BlockSpec((ROW_BLK, h), lambda i: (i, 0))
    bblk = pl.BlockSpec((ROW_BLK, 1), lambda i: (i, 0))
    full = lambda *shape: pl.BlockSpec(shape, lambda i: tuple(0 for _ in shape))
    h2 = w1.shape[1]
    h4 = w2.shape[1]
    return pl.pallas_call(
        _head_body,
        grid=(grid,),
        in_specs=[blk, blk, bblk,
                  full(2 * h, h2), full(h2), full(h2), full(h2),
                  full(h2, h4), full(h4), full(h4), full(h4),
                  full(h4, 1), full(1)],
        out_specs=pl.BlockSpec((NUM_GRAPHS_C, 1), lambda i: (0, 0)),
        out_shape=jax.ShapeDtypeStruct((NUM_GRAPHS_C, 1), jnp.float32),
        scratch_shapes=[pltpu.VMEM((NUM_GRAPHS_C, 2 * h), jnp.float32)],
    )(hre, him, batch.reshape(n, 1), w1, b1, g1, be1, w2, b2, g2, be2, w3, b3)


def _spmv_complex(u_row, u_col, lre, lim_, xre, xim, n):
    xr = xre[u_col]
    xi = xim[u_col]
    mre = lre[:, None] * xr - lim_[:, None] * xi
    mim = lre[:, None] * xi + lim_[:, None] * xr
    yre = jnp.zeros((n, xre.shape[1]), xre.dtype).at[u_row].add(mre)
    yim = jnp.zeros((n, xre.shape[1]), xre.dtype).at[u_row].add(mim)
    return yre, yim


CHUNK = 128
LANES = 16
NSUB = 16
ZROWS = 80


def _lane_bcast(v, lane):
    idx = jnp.full((LANES, 1), lane, jnp.int32)
    dn = lax.GatherDimensionNumbers(offset_dims=(), collapsed_slice_dims=(0,),
                                    start_index_map=(0,))
    return lax.gather(v, idx, dn, (1,),
                      mode=lax.GatherScatterMode.PROMISE_IN_BOUNDS)


def _spmv_sc(xre, xim, ucol, urow, lre, lim_, n):
    m = ucol.shape[0]
    h = xre.shape[1]
    nchunks = m // CHUNK
    ngroups = h // LANES
    mesh = plsc.VectorSubcoreMesh(core_axis_name="c", subcore_axis_name="s")

    @functools.partial(
        pl.kernel,
        mesh=mesh,
        out_type=[jax.ShapeDtypeStruct((n, h), jnp.float32),
                  jax.ShapeDtypeStruct((n, h), jnp.float32)],
        scratch_types=[
            pltpu.VMEM((CHUNK,), jnp.int32),
            pltpu.VMEM((1, CHUNK), jnp.int32),
            pltpu.VMEM((CHUNK,), jnp.float32),
            pltpu.VMEM((CHUNK,), jnp.float32),
            pltpu.VMEM((CHUNK, h), jnp.float32),
            pltpu.VMEM((CHUNK, h), jnp.float32),
            pltpu.VMEM_SHARED((n, h), jnp.float32),
        ],
    )
    def k(xre_hbm, xim_hbm, ucol_hbm, urow_hbm, lre_hbm, lim_hbm,
          yre_hbm, yim_hbm,
          gidx_v, sidx_v, la_v, lb_v, rr_v, ri_v, y_sh):
        core = lax.axis_index("c")
        sub = lax.axis_index("s")
        is_re = core == 0
        cf = lax.convert_element_type(1 - core, jnp.float32)
        cfv = jnp.full((LANES,), cf)
        ofv = 1.0 - cfv

        zv = jnp.zeros((LANES,), jnp.float32)

        def _zrow(r, _):
            for g in range(ngroups):
                rr_v[r, pl.ds(g * LANES, LANES)] = zv
            return 0

        lax.fori_loop(0, CHUNK, _zrow, 0)
        nblk = n // ZROWS
        nround = (nblk + NSUB - 1) // NSUB
        for bb in range(nround):
            blk = sub + bb * NSUB

            @pl.when(blk < nblk)
            def _():
                r0 = pl.multiple_of(blk * ZROWS, ZROWS)
                pltpu.sync_copy(rr_v.at[pl.ds(0, ZROWS)],
                                y_sh.at[pl.ds(r0, ZROWS)])
        plsc.subcore_barrier()

        def _chunk(kk, _):
            c = sub + kk * NSUB

            @pl.when(c < nchunks)
            def _():
                base = pl.multiple_of(c * CHUNK, CHUNK)
                pltpu.sync_copy(ucol_hbm.at[pl.ds(base, CHUNK)], gidx_v)
                pltpu.sync_copy(urow_hbm.at[pl.ds(base, CHUNK)],
                                sidx_v.at[0])
                pltpu.sync_copy(lre_hbm.at[pl.ds(base, CHUNK)], la_v)
                pltpu.sync_copy(lim_hbm.at[pl.ds(base, CHUNK)], lb_v)
                pltpu.sync_copy(xre_hbm.at[gidx_v], rr_v)
                pltpu.sync_copy(xim_hbm.at[gidx_v], ri_v)
                for mb in range(CHUNK // LANES):
                    la = la_v[pl.ds(mb * LANES, LANES)]
                    lb = lb_v[pl.ds(mb * LANES, LANES)]
                    av = la * cfv + lb * ofv
                    bv = la * ofv - lb * cfv
                    for e0 in range(LANES):
                        e = mb * LANES + e0
                        a_bc = _lane_bcast(av, e0)
                        b_bc = _lane_bcast(bv, e0)
                        for g in range(ngroups):
                            xr = rr_v[e, pl.ds(g * LANES, LANES)]
                            xi = ri_v[e, pl.ds(g * LANES, LANES)]
                            rr_v[e, pl.ds(g * LANES, LANES)] = (
                                a_bc * xr + b_bc * xi)
                pltpu.sync_copy(rr_v, y_sh.at[sidx_v.at[0]], add=True)
            return 0

        lax.fori_loop(0, (nchunks + NSUB - 1) // NSUB, _chunk, 0)
        plsc.subcore_barrier()

        for bb in range(nround):
            blk = sub + bb * NSUB

            @pl.when(blk < nblk)
            def _():
                r0 = pl.multiple_of(blk * ZROWS, ZROWS)
                pltpu.sync_copy(y_sh.at[pl.ds(r0, ZROWS)],
                                rr_v.at[pl.ds(0, ZROWS)])

                @pl.when(is_re)
                def _():
                    pltpu.sync_copy(rr_v.at[pl.ds(0, ZROWS)],
                                    yre_hbm.at[pl.ds(r0, ZROWS)])

                @pl.when(jnp.logical_not(is_re))
                def _():
                    pltpu.sync_copy(rr_v.at[pl.ds(0, ZROWS)],
                                    yim_hbm.at[pl.ds(r0, ZROWS)])

    return k(xre, xim, ucol, urow, lre, lim_)


def kernel(x, edge_index, edge_attr, batch, emb0, emb1, emb2, emb3, emb4,
           emb5, emb6, conv_w, conv_b, w1, b1, g1, be1, w2, b2, g2, be2,
           w3, b3):
    n = x.shape[0]
    embs = [emb0, emb1, emb2, emb3, emb4, emb5, emb6]

    row, col = edge_index[0], edge_index[1]
    ids = jnp.concatenate([row * n + col, col * n + row])
    ew = edge_attr[:, 0] + edge_attr[:, 1]
    sym_w = jnp.concatenate([ew, ew]) * 0.5
    theta_w = jnp.concatenate([ew, -ew]) * (2.0 * np.pi * Q_C)
    ids_s, sym_s, theta_s = lax.sort((ids, sym_w, theta_w), num_keys=1)
    u_row = (ids_s // n).astype(jnp.int32)
    u_col = (ids_s % n).astype(jnp.int32)
    neq = ids_s[1:] != ids_s[:-1]
    first = jnp.concatenate([jnp.ones((1,), bool), neq])
    last = jnp.concatenate([neq, jnp.ones((1,), bool)])

    def _segscan(vals, firsts):
        def comb(a, b):
            af, av = a
            bf, bv = b
            return af | bf, jnp.where(bf, bv, av + bv)
        _, out = lax.associative_scan(comb, (firsts, vals))
        return out

    S = _segscan(sym_s, first)
    T = _segscan(theta_s, first)
    Sl = jnp.where(last, S, 0.0)
    Tl = jnp.where(last, T, 0.0)
    rneq = u_row[1:] != u_row[:-1]
    rfirst = jnp.concatenate([jnp.ones((1,), bool), rneq])
    rlast = jnp.concatenate([rneq, jnp.ones((1,), bool)])
    dscan = _segscan(jnp.where(last, jnp.abs(S), 0.0), rfirst)
    deg = jnp.zeros((n,), jnp.float32).at[u_row].add(
        jnp.where(rlast, dscan, 0.0))
    safe = jnp.where(deg > 0.0, deg, 1.0)
    dinv = jnp.where(deg > 0.0, 1.0 / jnp.sqrt(safe), 0.0)
    lre = -Sl * jnp.cos(Tl)
    lim_ = -Sl * jnp.sin(Tl)
    dinv_c = dinv[:, None]

    h = jnp.concatenate([embs[i][x[:, i]] for i in range(7)], axis=1)
    h = jnp.pad(h, ((0, 0), (0, HPAD - HIDDEN_C)))
    hre, him = h, h

    wp = jnp.pad(conv_w, ((0, 0), (0, 0), (0, HPAD - HIDDEN_C), (0, HPAD - HIDDEN_C)))
    bp = jnp.pad(conv_b, ((0, 0), (0, HPAD - HIDDEN_C)))
    w1p = jnp.zeros((2 * HPAD, w1.shape[1]), jnp.float32)
    w1p = w1p.at[:HIDDEN_C].set(w1[:HIDDEN_C])
    w1p = w1p.at[HPAD:HPAD + HIDDEN_C].set(w1[HIDDEN_C:])

    for l in range(conv_w.shape[0]):
        t0re, t0im = hre, him
        r1, i1 = _spmv_sc(dinv_c * t0re, dinv_c * t0im, u_col, u_row,
                          lre, lim_, n)
        t1re, t1im = dinv_c * r1, dinv_c * i1
        r2, i2 = _spmv_sc(dinv_c * t1re, dinv_c * t1im, u_col, u_row,
                          lre, lim_, n)
        t2re = 2.0 * (dinv_c * r2) - t0re
        t2im = 2.0 * (dinv_c * i2) - t0im
        hre, him = _conv_layer(t0re, t1re, t2re, t0im, t1im, t2im,
                               wp[l], bp[l])

    return _head(hre, him, batch, w1p, b1, g1, be1, w2, b2, g2, be2, w3, b3)


# speedup vs baseline: 2.1567x; 1.2721x over previous
import functools

import jax
import jax.numpy as jnp
import numpy as np
from jax import lax
from jax.experimental import pallas as pl
from jax.experimental.pallas import tpu as pltpu
from jax.experimental.pallas import tpu_sc as plsc

N_NODES_C = 10000
HIDDEN_C = 112
NUM_GRAPHS_C = 128
EPS_C = 1e-5
Q_C = 0.25

ROW_BLK = 1000
HPAD = 128


def _conv_layer_body(t0re_ref, t1re_ref, t2re_ref, t0im_ref, t1im_ref, t2im_ref,
                     w_ref, b_ref, outre_ref, outim_ref):
    w0 = w_ref[0]
    w1 = w_ref[1]
    w2 = w_ref[2]
    b = b_ref[...]
    outre = (jnp.dot(t0re_ref[...], w0, preferred_element_type=jnp.float32)
             + jnp.dot(t1re_ref[...], w1, preferred_element_type=jnp.float32)
             + jnp.dot(t2re_ref[...], w2, preferred_element_type=jnp.float32)
             + b[None, :])
    outim = (jnp.dot(t0im_ref[...], w0, preferred_element_type=jnp.float32)
             + jnp.dot(t1im_ref[...], w1, preferred_element_type=jnp.float32)
             + jnp.dot(t2im_ref[...], w2, preferred_element_type=jnp.float32)
             + b[None, :])
    mask = (outre >= 0.0).astype(jnp.float32)
    outre_ref[...] = mask * outre
    outim_ref[...] = mask * outim


def _conv_layer(t0re, t1re, t2re, t0im, t1im, t2im, w, b):
    n = t0re.shape[0]
    h = t0re.shape[1]
    grid = n // ROW_BLK
    blk = pl.BlockSpec((ROW_BLK, h), lambda i: (i, 0))
    full_w = pl.BlockSpec((3, h, h), lambda i: (0, 0, 0))
    full_b = pl.BlockSpec((h,), lambda i: (0,))
    return pl.pallas_call(
        _conv_layer_body,
        grid=(grid,),
        in_specs=[blk, blk, blk, blk, blk, blk, full_w, full_b],
        out_specs=[blk, blk],
        out_shape=[jax.ShapeDtypeStruct((n, h), jnp.float32),
                   jax.ShapeDtypeStruct((n, h), jnp.float32)],
    )(t0re, t1re, t2re, t0im, t1im, t2im, w, b)


def _head_body(hre_ref, him_ref, batch_ref, w1_ref, b1_ref, g1_ref, be1_ref,
               w2_ref, b2_ref, g2_ref, be2_ref, w3_ref, b3_ref, out_ref, acc_ref):
    i = pl.program_id(0)
    ids = batch_ref[...]
    onehot = (ids == jax.lax.broadcasted_iota(jnp.int32, (1, NUM_GRAPHS_C), 1)
              ).astype(jnp.float32)
    hcat = jnp.concatenate([hre_ref[...], him_ref[...]], axis=1)
    part = jax.lax.dot_general(onehot, hcat, (((0,), (0,)), ((), ())),
                               preferred_element_type=jnp.float32,
                               precision=jax.lax.Precision.HIGHEST)

    @pl.when(i == 0)
    def _init():
        acc_ref[...] = jnp.zeros_like(acc_ref)

    acc_ref[...] += part

    @pl.when(i == pl.num_programs(0) - 1)
    def _fin():
        pooled = acc_ref[...]

        def bn(z, g, be):
            mu = jnp.mean(z, axis=0, keepdims=True)
            var = jnp.mean((z - mu) ** 2, axis=0, keepdims=True)
            return (z - mu) / jnp.sqrt(var + EPS_C) * g[None, :] + be[None, :]

        z = jnp.dot(pooled, w1_ref[...], preferred_element_type=jnp.float32) + b1_ref[...][None, :]
        z = jax.nn.relu(bn(z, g1_ref[...], be1_ref[...]))
        z = jnp.dot(z, w2_ref[...], preferred_element_type=jnp.float32) + b2_ref[...][None, :]
        z = jax.nn.relu(bn(z, g2_ref[...], be2_ref[...]))
        out_ref[...] = jnp.dot(z, w3_ref[...], preferred_element_type=jnp.float32) + b3_ref[...][None, :]


def _head(hre, him, batch, w1, b1, g1, be1, w2, b2, g2, be2, w3, b3):
    n, h = hre.shape
    grid = n // ROW_BLK
    blk = pl.BlockSpec((ROW_BLK, h), lambda i: (i, 0))
    bblk = pl.BlockSpec((ROW_BLK, 1), lambda i: (i, 0))
    full = lambda *shape: pl.BlockSpec(shape, lambda i: tuple(0 for _ in shape))
    h2 = w1.shape[1]
    h4 = w2.shape[1]
    return pl.pallas_call(
        _head_body,
        grid=(grid,),
        in_specs=[blk, blk, bblk,
                  full(2 * h, h2), full(h2), full(h2), full(h2),
                  full(h2, h4), full(h4), full(h4), full(h4),
                  full(h4, 1), full(1)],
        out_specs=pl.BlockSpec((NUM_GRAPHS_C, 1), lambda i: (0, 0)),
        out_shape=jax.ShapeDtypeStruct((NUM_GRAPHS_C, 1), jnp.float32),
        scratch_shapes=[pltpu.VMEM((NUM_GRAPHS_C, 2 * h), jnp.float32)],
    )(hre, him, batch.reshape(n, 1), w1, b1, g1, be1, w2, b2, g2, be2, w3, b3)


def _spmv_complex(u_row, u_col, lre, lim_, xre, xim, n):
    xr = xre[u_col]
    xi = xim[u_col]
    mre = lre[:, None] * xr - lim_[:, None] * xi
    mim = lre[:, None] * xi + lim_[:, None] * xr
    yre = jnp.zeros((n, xre.shape[1]), xre.dtype).at[u_row].add(mre)
    yim = jnp.zeros((n, xre.shape[1]), xre.dtype).at[u_row].add(mim)
    return yre, yim


CHUNK = 32
LANES = 16
NSUB = 16
ZROWS = 80


def _lane_bcast(v, lane):
    idx = jnp.full((LANES, 1), lane, jnp.int32)
    dn = lax.GatherDimensionNumbers(offset_dims=(), collapsed_slice_dims=(0,),
                                    start_index_map=(0,))
    return lax.gather(v, idx, dn, (1,),
                      mode=lax.GatherScatterMode.PROMISE_IN_BOUNDS)


def _spmv_sc(xre, xim, idp, valp, n):
    nchunks = idp.shape[0]
    h = xre.shape[1]
    ngroups = h // LANES
    cpt = nchunks // NSUB
    mesh = plsc.VectorSubcoreMesh(core_axis_name="c", subcore_axis_name="s")

    @functools.partial(
        pl.kernel,
        mesh=mesh,
        out_type=[jax.ShapeDtypeStruct((n, h), jnp.float32),
                  jax.ShapeDtypeStruct((n, h), jnp.float32)],
        scratch_types=[
            pltpu.VMEM((2, 2, CHUNK), jnp.int32),
            pltpu.VMEM((2, 2, CHUNK), jnp.float32),
            pltpu.VMEM((2, CHUNK), jnp.int32),
            pltpu.VMEM((2, CHUNK, h), jnp.float32),
            pltpu.VMEM((2, CHUNK, h), jnp.float32),
            pltpu.VMEM((2, CHUNK, h), jnp.float32),
            pltpu.VMEM((ZROWS, h), jnp.float32),
            pltpu.VMEM_SHARED((n, h), jnp.float32),
            pltpu.SemaphoreType.DMA,
            pltpu.SemaphoreType.DMA,
            pltpu.SemaphoreType.DMA,
            pltpu.SemaphoreType.DMA,
            pltpu.SemaphoreType.DMA,
            pltpu.SemaphoreType.DMA,
        ],
    )
    def k(xre_hbm, xim_hbm, idp_hbm, valp_hbm, yre_hbm, yim_hbm,
          ids_v, vals_v, sidx_v, rr_v, ri_v, out_v, stage_v, y_sh,
          isem0, isem1, gsem0, gsem1, ssem0, ssem1):
        core = lax.axis_index("c")
        sub = lax.axis_index("s")
        is_re = core == 0
        cf = lax.convert_element_type(1 - core, jnp.float32)
        cfv = jnp.full((LANES,), cf)
        ofv = 1.0 - cfv
        isem = (isem0, isem1)
        gsem = (gsem0, gsem1)
        ssem = (ssem0, ssem1)
        cbase = sub * cpt

        zv = jnp.zeros((LANES,), jnp.float32)

        def _zrow(r, _):
            for g in range(ngroups):
                stage_v[r, pl.ds(g * LANES, LANES)] = zv
            return 0

        lax.fori_loop(0, ZROWS, _zrow, 0)
        nblk = n // ZROWS
        nround = (nblk + NSUB - 1) // NSUB
        for bb in range(nround):
            blk = sub + bb * NSUB

            @pl.when(blk < nblk)
            def _():
                r0 = pl.multiple_of(blk * ZROWS, ZROWS)
                pltpu.sync_copy(stage_v, y_sh.at[pl.ds(r0, ZROWS)])
        plsc.subcore_barrier()

        def issue_ids(c, b):
            pltpu.async_copy(idp_hbm.at[c], ids_v.at[b], isem[b])
            pltpu.async_copy(valp_hbm.at[c], vals_v.at[b], isem[b])

        def wait_ids(b):
            pltpu.make_async_copy(idp_hbm.at[0], ids_v.at[b], isem[b]).wait()
            pltpu.make_async_copy(valp_hbm.at[0], vals_v.at[b], isem[b]).wait()

        def issue_gather(b):
            pltpu.async_copy(xre_hbm.at[ids_v.at[b, 0]], rr_v.at[b], gsem[b])
            pltpu.async_copy(xim_hbm.at[ids_v.at[b, 0]], ri_v.at[b], gsem[b])

        def wait_gather(b):
            pltpu.make_async_copy(xre_hbm.at[ids_v.at[b, 0]], rr_v.at[b],
                                  gsem[b]).wait()
            pltpu.make_async_copy(xim_hbm.at[ids_v.at[b, 0]], ri_v.at[b],
                                  gsem[b]).wait()

        def issue_scatter(b):
            pltpu.async_copy(out_v.at[b], y_sh.at[sidx_v.at[b]], ssem[b],
                             add=True)

        def wait_scatter(b):
            pltpu.make_async_copy(out_v.at[b], y_sh.at[sidx_v.at[b]],
                                  ssem[b]).wait()

        def compute(b):
            for mb in range(CHUNK // LANES):
                sl = pl.ds(mb * LANES, LANES)
                sidx_v[b, sl] = ids_v[b, 1, sl]
                la = vals_v[b, 0, sl]
                lb = vals_v[b, 1, sl]
                av = la * cfv + lb * ofv
                bv = la * ofv - lb * cfv
                for e0 in range(LANES):
                    e = mb * LANES + e0
                    a_bc = _lane_bcast(av, e0)
                    b_bc = _lane_bcast(bv, e0)
                    for g in range(ngroups):
                        gs = pl.ds(g * LANES, LANES)
                        out_v[b, e, gs] = a_bc * rr_v[b, e, gs] + b_bc * ri_v[b, e, gs]

        issue_ids(cbase, 0)
        issue_ids(cbase + 1, 1)
        wait_ids(0)
        issue_gather(0)

        def pair(p, _):
            for b in range(2):
                kk = p * 2 + b

                @pl.when(kk < cpt)
                def _():
                    wait_gather(b)

                    @pl.when(kk >= 2)
                    def _():
                        wait_scatter(b)

                    @pl.when(kk + 2 < cpt)
                    def _():
                        issue_ids(cbase + kk + 2, b)

                    compute(b)
                    issue_scatter(b)

                    @pl.when(kk + 1 < cpt)
                    def _():
                        wait_ids(1 - b)
                        issue_gather(1 - b)
            return 0

        lax.fori_loop(0, (cpt + 1) // 2, pair, 0)
        wait_scatter((cpt - 2) % 2)
        wait_scatter((cpt - 1) % 2)
        plsc.subcore_barrier()

        for bb in range(nround):
            blk = sub + bb * NSUB

            @pl.when(blk < nblk)
            def _():
                r0 = pl.multiple_of(blk * ZROWS, ZROWS)
                pltpu.sync_copy(y_sh.at[pl.ds(r0, ZROWS)], stage_v)

                @pl.when(is_re)
                def _():
                    pltpu.sync_copy(stage_v, yre_hbm.at[pl.ds(r0, ZROWS)])

                @pl.when(jnp.logical_not(is_re))
                def _():
                    pltpu.sync_copy(stage_v, yim_hbm.at[pl.ds(r0, ZROWS)])

    return k(xre, xim, idp, valp)


def kernel(x, edge_index, edge_attr, batch, emb0, emb1, emb2, emb3, emb4,
           emb5, emb6, conv_w, conv_b, w1, b1, g1, be1, w2, b2, g2, be2,
           w3, b3):
    n = x.shape[0]
    embs = [emb0, emb1, emb2, emb3, emb4, emb5, emb6]

    row, col = edge_index[0], edge_index[1]
    ids = jnp.concatenate([row * n + col, col * n + row])
    ew = edge_attr[:, 0] + edge_attr[:, 1]
    sym_w = jnp.concatenate([ew, ew]) * 0.5
    theta_w = jnp.concatenate([ew, -ew]) * (2.0 * np.pi * Q_C)
    ids_s, sym_s, theta_s = lax.sort((ids, sym_w, theta_w), num_keys=1)
    u_row = (ids_s // n).astype(jnp.int32)
    u_col = (ids_s % n).astype(jnp.int32)
    neq = ids_s[1:] != ids_s[:-1]
    first = jnp.concatenate([jnp.ones((1,), bool), neq])
    last = jnp.concatenate([neq, jnp.ones((1,), bool)])

    def _segscan(vals, firsts):
        def comb(a, b):
            af, av = a
            bf, bv = b
            return af | bf, jnp.where(bf, bv, av + bv)
        _, out = lax.associative_scan(comb, (firsts, vals))
        return out

    S = _segscan(sym_s, first)
    T = _segscan(theta_s, first)
    Sl = jnp.where(last, S, 0.0)
    Tl = jnp.where(last, T, 0.0)
    rneq = u_row[1:] != u_row[:-1]
    rfirst = jnp.concatenate([jnp.ones((1,), bool), rneq])
    rlast = jnp.concatenate([rneq, jnp.ones((1,), bool)])
    dscan = _segscan(jnp.where(last, jnp.abs(S), 0.0), rfirst)
    deg = jnp.zeros((n,), jnp.float32).at[u_row].add(
        jnp.where(rlast, dscan, 0.0))
    safe = jnp.where(deg > 0.0, deg, 1.0)
    dinv = jnp.where(deg > 0.0, 1.0 / jnp.sqrt(safe), 0.0)
    lre = -Sl * jnp.cos(Tl)
    lim_ = -Sl * jnp.sin(Tl)
    dinv_c = dinv[:, None]
    nch = u_col.shape[0] // CHUNK
    idp = jnp.stack([u_col.reshape(nch, CHUNK), u_row.reshape(nch, CHUNK)],
                    axis=1)
    valp = jnp.stack([lre.reshape(nch, CHUNK), lim_.reshape(nch, CHUNK)],
                     axis=1)

    h = jnp.concatenate([embs[i][x[:, i]] for i in range(7)], axis=1)
    h = jnp.pad(h, ((0, 0), (0, HPAD - HIDDEN_C)))
    hre, him = h, h

    wp = jnp.pad(conv_w, ((0, 0), (0, 0), (0, HPAD - HIDDEN_C), (0, HPAD - HIDDEN_C)))
    bp = jnp.pad(conv_b, ((0, 0), (0, HPAD - HIDDEN_C)))
    w1p = jnp.zeros((2 * HPAD, w1.shape[1]), jnp.float32)
    w1p = w1p.at[:HIDDEN_C].set(w1[:HIDDEN_C])
    w1p = w1p.at[HPAD:HPAD + HIDDEN_C].set(w1[HIDDEN_C:])

    for l in range(conv_w.shape[0]):
        t0re, t0im = hre, him
        r1, i1 = _spmv_sc(dinv_c * t0re, dinv_c * t0im, idp, valp, n)
        t1re, t1im = dinv_c * r1, dinv_c * i1
        r2, i2 = _spmv_sc(dinv_c * t1re, dinv_c * t1im, idp, valp, n)
        t2re = 2.0 * (dinv_c * r2) - t0re
        t2im = 2.0 * (dinv_c * i2) - t0im
        hre, him = _conv_layer(t0re, t1re, t2re, t0im, t1im, t2im,
                               wp[l], bp[l])

    return _head(hre, him, batch, w1p, b1, g1, be1, w2, b2, g2, be2, w3, b3)
